# baseline (speedup 1.0000x reference)
import functools

import jax
import jax.numpy as jnp
from jax import lax
from jax.experimental import pallas as pl
from jax.experimental.pallas import tpu as pltpu
from jax.experimental.pallas import tpu_sc as plsc

EPS = 1e-5
NSPLIT = 5


def _softplus(x):
    return jnp.maximum(x, 0.0) + jnp.log(1.0 + jnp.exp(-jnp.abs(x)))


def _pack_bf16(lo_f, hi_f):
    lo = lax.bitcast_convert_type(lo_f, jnp.int32)
    hi = lax.bitcast_convert_type(hi_f, jnp.int32)
    lo_r = lo + jnp.int32(0x7FFF) + jnp.bitwise_and(
        lax.shift_right_logical(lo, 16), jnp.int32(1))
    hi_r = hi + jnp.int32(0x7FFF) + jnp.bitwise_and(
        lax.shift_right_logical(hi, 16), jnp.int32(1))
    return jnp.bitwise_or(lax.shift_right_logical(lo_r, 16),
                          jnp.bitwise_and(hi_r, jnp.int32(-65536)))


def _unpack_lo(packed):
    return lax.bitcast_convert_type(jnp.left_shift(packed, 16), jnp.float32)


def _unpack_hi(packed):
    return lax.bitcast_convert_type(
        jnp.bitwise_and(packed, jnp.int32(-65536)), jnp.float32)


def _prep_y_body(x_ref, w_ref, y_ref, *, d):
    y = jnp.dot(x_ref[...], w_ref[...], preferred_element_type=jnp.float32)
    y_ref[...] = _pack_bf16(y[:, :d], y[:, d:])


def _transpose_body(idx_ref, idxt_ref):
    idxt_ref[...] = idx_ref[...].T


def _prep_a_body(x_ref, w_ref, b_ref, a_ref):
    a_ref[...] = jnp.dot(x_ref[...], w_ref[...],
                         preferred_element_type=jnp.float32) + b_ref[...]


def _row_block(yg_ref, a_ref, ef_ref, we_ref, d):
    e = jnp.dot(ef_ref[...], we_ref[...], preferred_element_type=jnp.float32)
    yg = yg_ref[...]
    rlo = _unpack_lo(yg) + e[:, :d] + a_ref[:, :d]
    rhi = _unpack_hi(yg) + e[:, d:] + a_ref[:, d:]
    return rlo, rhi


def _stats_body(yg_ref, a_ref, ef_ref, we_ref, sum_ref, sq_ref, *, bn, d):
    @pl.when(pl.program_id(0) == 0)
    def _init():
        sum_ref[...] = jnp.zeros_like(sum_ref)
        sq_ref[...] = jnp.zeros_like(sq_ref)

    rlo, rhi = _row_block(yg_ref, a_ref, ef_ref, we_ref, d)
    ones_r = jnp.ones((1, bn), jnp.float32)
    dot = functools.partial(jnp.dot, preferred_element_type=jnp.float32)
    sum_ref[:, :d] += dot(ones_r, rlo)
    sum_ref[:, d:] += dot(ones_r, rhi)
    sq_ref[:, :d] += dot(ones_r, rlo * rlo)
    sq_ref[:, d:] += dot(ones_r, rhi * rhi)


def _pass2_body(yg_ref, a_ref, ef_ref, we_ref, s1_ref, t1_ref, sprev_ref,
                s_ref, sum_ref, sq_ref, *, bn, m, d):
    j = pl.program_id(0)

    rlo, rhi = _row_block(yg_ref, a_ref, ef_ref, we_ref, d)
    rnlo = rlo * s1_ref[:, :d] + t1_ref[:, :d]
    rnhi = rhi * s1_ref[:, d:] + t1_ref[:, d:]
    prod = jax.nn.sigmoid(rnlo) * _softplus(rnhi)

    @pl.when(j == 0)
    def _first():
        s_ref[...] = prod

    @pl.when(j != 0)
    def _acc():
        s_ref[...] += prod

    @pl.when(j == m - 1)
    def _stats():
        s = s_ref[...]
        ones_n = jnp.ones((1, bn), jnp.float32)
        dot = functools.partial(jnp.dot, preferred_element_type=jnp.float32)
        sum_ref[...] = dot(ones_n, s)
        sq_ref[...] = dot(ones_n, s * s)


def _final_body(x_ref, s_ref, s2_ref, t2_ref, o_ref):
    o_ref[...] = _softplus(x_ref[...] + s_ref[...] * s2_ref[...] + t2_ref[...])


def _make_gather(nrows, dcols, nw):
    rows_per_w = nrows // nw
    ch = 40
    ksub = 5
    sup = ch * ksub
    nsup = rows_per_w // sup
    nch = rows_per_w // ch
    mesh = plsc.VectorSubcoreMesh(core_axis_name="c", subcore_axis_name="s")

    @functools.partial(
        pl.kernel, mesh=mesh,
        out_type=jax.ShapeDtypeStruct((nrows, dcols), jnp.int32),
        scratch_types=[
            pltpu.VMEM((nch, ch), jnp.int32),
            pltpu.VMEM((sup, dcols), jnp.int32),
            pltpu.VMEM((sup, dcols), jnp.int32),
            pltpu.SemaphoreType.DMA,
            pltpu.SemaphoreType.DMA,
        ],
    )
    def gk(table_hbm, idx_hbm, out_hbm, idx_v, buf0, buf1, sem0, sem1):
        ncores = 2
        wid = lax.axis_index("s") * ncores + lax.axis_index("c")
        base = wid * rows_per_w
        pltpu.sync_copy(idx_hbm.at[wid], idx_v)

        def issue(sc, buf, sem):
            for k in range(ksub):
                pltpu.async_copy(table_hbm.at[idx_v.at[sc * ksub + k]],
                                 buf.at[pl.ds(k * ch, ch)], sem)

        def drain_store(sc, buf, sem):
            for k in range(ksub):
                pltpu.make_async_copy(table_hbm.at[pl.ds(0, ch)],
                                      buf.at[pl.ds(k * ch, ch)], sem).wait()
            off = pl.multiple_of(base + sc * sup, 8)
            pltpu.sync_copy(buf, out_hbm.at[pl.ds(off, sup)])

        issue(0, buf0, sem0)

        def body(p, carry):
            s0 = 2 * p
            issue(s0 + 1, buf1, sem1)
            drain_store(s0, buf0, sem0)
            issue(s0 + 2, buf0, sem0)
            drain_store(s0 + 1, buf1, sem1)
            return carry

        lax.fori_loop(0, nsup // 2, body, 0)
        drain_store(nsup - 1, buf0, sem0)

    return gk


def kernel(input_features, num_features, feature_index, W, b,
           bn1_gamma, bn1_beta, bn2_gamma, bn2_beta):
    n, d = input_features.shape
    m = feature_index.shape[1]
    de = num_features.shape[2]
    d2 = 2 * d
    nm = n * m
    nc = n // NSPLIT
    rows_c = nc * m

    wt_self = W[:, :d].T
    wt_nbr = W[:, d:2 * d].T
    wt_edge = W[:, 2 * d:].T
    bfull = b[None, :]
    idx32 = feature_index.astype(jnp.int32)
    ef = jnp.transpose(num_features, (1, 0, 2)).reshape(nm, de)

    bnp = 1000
    y_mat = pl.pallas_call(
        functools.partial(_prep_y_body, d=d),
        grid=(n // bnp,),
        in_specs=[
            pl.BlockSpec((bnp, d), lambda i: (i, 0)),
            pl.BlockSpec((d, d2), lambda i: (0, 0)),
        ],
        out_specs=pl.BlockSpec((bnp, d2 // 2), lambda i: (i, 0)),
        out_shape=jax.ShapeDtypeStruct((n, d2 // 2), jnp.int32),
    )(input_features, wt_nbr)
    idx_t = pl.pallas_call(
        _transpose_body,
        in_specs=[pl.BlockSpec((n, m), lambda: (0, 0))],
        out_specs=pl.BlockSpec((m, n), lambda: (0, 0)),
        out_shape=jax.ShapeDtypeStruct((m, n), jnp.int32),
    )(idx32)
    idx5 = idx_t.reshape(m, NSPLIT, 2, (nc // 2) // 40, 40)

    gather = _make_gather(rows_c, d2 // 2, 32)
    yg_parts = [
        gather(y_mat, idx5[:, c].reshape(32, (nc // 2) // 40, 40))
        for c in range(NSPLIT)
    ]

    a_mat = pl.pallas_call(
        _prep_a_body,
        grid=(n // bnp,),
        in_specs=[
            pl.BlockSpec((bnp, d), lambda i: (i, 0)),
            pl.BlockSpec((d, d2), lambda i: (0, 0)),
            pl.BlockSpec((1, d2), lambda i: (0, 0)),
        ],
        out_specs=pl.BlockSpec((bnp, d2), lambda i: (i, 0)),
        out_shape=jax.ShapeDtypeStruct((n, d2), jnp.float32),
    )(input_features, wt_self, bfull)

    bn = nc
    acc_spec2 = pl.BlockSpec((1, d2), lambda j: (0, 0))
    sums = []
    for c in range(NSPLIT):
        sums.append(pl.pallas_call(
            functools.partial(_stats_body, bn=bn, d=d),
            grid=(m,),
            in_specs=[
                pl.BlockSpec((bn, d2 // 2), lambda j: (j, 0)),
                pl.BlockSpec((bn, d2), lambda j, c=c: (c, 0)),
                pl.BlockSpec((bn, de), lambda j, c=c: (j * NSPLIT + c, 0)),
                pl.BlockSpec((de, d2), lambda j: (0, 0)),
            ],
            out_specs=[acc_spec2, acc_spec2],
            out_shape=[
                jax.ShapeDtypeStruct((1, d2), jnp.float32),
                jax.ShapeDtypeStruct((1, d2), jnp.float32),
            ],
        )(yg_parts[c], a_mat, ef, wt_edge))

    s1_sum = sum(s for s, _ in sums)
    s1_sq = sum(q for _, q in sums)
    mean1 = s1_sum / nm
    var1 = s1_sq / nm - mean1 * mean1
    inv1 = lax.rsqrt(var1 + EPS)
    scale1 = bn1_gamma[None, :] * inv1
    shift1 = bn1_beta[None, :] - mean1 * scale1

    s_mat = jnp.zeros((n, d), jnp.float32)
    acc_spec1 = pl.BlockSpec((1, d), lambda j: (0, 0))
    s2_parts = []
    for c in range(NSPLIT):
        s_mat, p_sum, p_sq = pl.pallas_call(
            functools.partial(_pass2_body, bn=bn, m=m, d=d),
            grid=(m,),
            in_specs=[
                pl.BlockSpec((bn, d2 // 2), lambda j: (j, 0)),
                pl.BlockSpec((bn, d2), lambda j, c=c: (c, 0)),
                pl.BlockSpec((bn, de), lambda j, c=c: (j * NSPLIT + c, 0)),
                pl.BlockSpec((de, d2), lambda j: (0, 0)),
                pl.BlockSpec((1, d2), lambda j: (0, 0)),
                pl.BlockSpec((1, d2), lambda j: (0, 0)),
                pl.BlockSpec((bn, d), lambda j, c=c: (c, 0)),
            ],
            out_specs=[pl.BlockSpec((bn, d), lambda j, c=c: (c, 0)),
                       acc_spec1, acc_spec1],
            out_shape=[
                jax.ShapeDtypeStruct((n, d), jnp.float32),
                jax.ShapeDtypeStruct((1, d), jnp.float32),
                jax.ShapeDtypeStruct((1, d), jnp.float32),
            ],
            input_output_aliases={6: 0},
        )(yg_parts[c], a_mat, ef, wt_edge, scale1, shift1, s_mat)
        s2_parts.append((p_sum, p_sq))

    s2_sum = sum(s for s, _ in s2_parts)
    s2_sq = sum(q for _, q in s2_parts)
    mean2 = s2_sum / n
    var2 = s2_sq / n - mean2 * mean2
    inv2 = lax.rsqrt(var2 + EPS)
    scale2 = bn2_gamma[None, :] * inv2
    shift2 = bn2_beta[None, :] - mean2 * scale2

    bnf = 1000
    out = pl.pallas_call(
        _final_body,
        grid=(n // bnf,),
        in_specs=[
            pl.BlockSpec((bnf, d), lambda i: (i, 0)),
            pl.BlockSpec((bnf, d), lambda i: (i, 0)),
            pl.BlockSpec((1, d), lambda i: (0, 0)),
            pl.BlockSpec((1, d), lambda i: (0, 0)),
        ],
        out_specs=pl.BlockSpec((bnf, d), lambda i: (i, 0)),
        out_shape=jax.ShapeDtypeStruct((n, d), jnp.float32),
    )(input_features, s_mat, scale2, shift2)
    return out


# speedup vs baseline: 1.0107x; 1.0107x over previous
import functools

import jax
import jax.numpy as jnp
from jax import lax
from jax.experimental import pallas as pl
from jax.experimental.pallas import tpu as pltpu
from jax.experimental.pallas import tpu_sc as plsc

EPS = 1e-5


def _softplus(x):
    return jnp.maximum(x, 0.0) + jnp.log(1.0 + jnp.exp(-jnp.abs(x)))


def _pack_bf16(lo_f, hi_f):
    lo = lax.bitcast_convert_type(lo_f, jnp.int32)
    hi = lax.bitcast_convert_type(hi_f, jnp.int32)
    lo_r = lo + jnp.int32(0x7FFF) + jnp.bitwise_and(
        lax.shift_right_logical(lo, 16), jnp.int32(1))
    hi_r = hi + jnp.int32(0x7FFF) + jnp.bitwise_and(
        lax.shift_right_logical(hi, 16), jnp.int32(1))
    return jnp.bitwise_or(lax.shift_right_logical(lo_r, 16),
                          jnp.bitwise_and(hi_r, jnp.int32(-65536)))


def _unpack_lo(packed):
    return lax.bitcast_convert_type(jnp.left_shift(packed, 16), jnp.float32)


def _unpack_hi(packed):
    return lax.bitcast_convert_type(
        jnp.bitwise_and(packed, jnp.int32(-65536)), jnp.float32)


def _prep_y_body(x_ref, w_ref, y_ref, *, d):
    y = jnp.dot(x_ref[...], w_ref[...], preferred_element_type=jnp.float32)
    y_ref[...] = _pack_bf16(y[:, :d], y[:, d:])


def _transpose_body(idx_ref, idxt_ref):
    idxt_ref[...] = idx_ref[...].T


def _prep_a_body(x_ref, w_ref, b_ref, a_ref):
    a_ref[...] = jnp.dot(x_ref[...], w_ref[...],
                         preferred_element_type=jnp.float32) + b_ref[...]


def _row_block(yg_ref, a_ref, ef_ref, we_ref, d):
    e = jnp.dot(ef_ref[...], we_ref[...], preferred_element_type=jnp.float32)
    yg = yg_ref[...]
    rlo = _unpack_lo(yg) + e[:, :d] + a_ref[:, :d]
    rhi = _unpack_hi(yg) + e[:, d:] + a_ref[:, d:]
    return rlo, rhi


def _stats_body(yg_ref, a_ref, ef_ref, we_ref, sum_ref, sq_ref, *, bn, d):
    @pl.when((pl.program_id(0) == 0) & (pl.program_id(1) == 0))
    def _init():
        sum_ref[...] = jnp.zeros_like(sum_ref)
        sq_ref[...] = jnp.zeros_like(sq_ref)

    rlo, rhi = _row_block(yg_ref, a_ref, ef_ref, we_ref, d)
    ones_r = jnp.ones((1, bn), jnp.float32)
    dot = functools.partial(jnp.dot, preferred_element_type=jnp.float32)
    sum_ref[:, :d] += dot(ones_r, rlo)
    sum_ref[:, d:] += dot(ones_r, rhi)
    sq_ref[:, :d] += dot(ones_r, rlo * rlo)
    sq_ref[:, d:] += dot(ones_r, rhi * rhi)


def _pass2_body(yg_ref, a_ref, ef_ref, we_ref, s1_ref, t1_ref,
                s_ref, sum_ref, sq_ref, *, bn, m, d):
    i = pl.program_id(0)
    j = pl.program_id(1)

    rlo, rhi = _row_block(yg_ref, a_ref, ef_ref, we_ref, d)
    rnlo = rlo * s1_ref[:, :d] + t1_ref[:, :d]
    rnhi = rhi * s1_ref[:, d:] + t1_ref[:, d:]
    prod = jax.nn.sigmoid(rnlo) * _softplus(rnhi)

    @pl.when(j == 0)
    def _first():
        s_ref[...] = prod

    @pl.when(j != 0)
    def _acc():
        s_ref[...] += prod

    @pl.when((i == 0) & (j == m - 1))
    def _init_stats():
        sum_ref[...] = jnp.zeros_like(sum_ref)
        sq_ref[...] = jnp.zeros_like(sq_ref)

    @pl.when(j == m - 1)
    def _stats():
        s = s_ref[...]
        ones_n = jnp.ones((1, bn), jnp.float32)
        dot = functools.partial(jnp.dot, preferred_element_type=jnp.float32)
        sum_ref[...] += dot(ones_n, s)
        sq_ref[...] += dot(ones_n, s * s)


def _final_body(x_ref, s_ref, s2_ref, t2_ref, o_ref):
    o_ref[...] = _softplus(x_ref[...] + s_ref[...] * s2_ref[...] + t2_ref[...])


def _make_gather(nrows, dcols, nw):
    rows_per_w = nrows // nw
    ch = 40
    ksub = 5
    sup = ch * ksub
    nsup = rows_per_w // sup
    nch = rows_per_w // ch
    mesh = plsc.VectorSubcoreMesh(core_axis_name="c", subcore_axis_name="s")

    @functools.partial(
        pl.kernel, mesh=mesh,
        out_type=jax.ShapeDtypeStruct((nrows, dcols), jnp.int32),
        scratch_types=[
            pltpu.VMEM((nch, ch), jnp.int32),
            pltpu.VMEM((sup, dcols), jnp.int32),
            pltpu.VMEM((sup, dcols), jnp.int32),
            pltpu.SemaphoreType.DMA,
            pltpu.SemaphoreType.DMA,
        ],
    )
    def gk(table_hbm, idx_hbm, out_hbm, idx_v, buf0, buf1, sem0, sem1):
        ncores = 2
        wid = lax.axis_index("s") * ncores + lax.axis_index("c")
        base = wid * rows_per_w
        pltpu.sync_copy(idx_hbm.at[wid], idx_v)

        def issue(sc, buf, sem):
            for k in range(ksub):
                pltpu.async_copy(table_hbm.at[idx_v.at[sc * ksub + k]],
                                 buf.at[pl.ds(k * ch, ch)], sem)

        def drain_store(sc, buf, sem):
            for k in range(ksub):
                pltpu.make_async_copy(table_hbm.at[pl.ds(0, ch)],
                                      buf.at[pl.ds(k * ch, ch)], sem).wait()
            off = pl.multiple_of(base + sc * sup, 8)
            pltpu.sync_copy(buf, out_hbm.at[pl.ds(off, sup)])

        issue(0, buf0, sem0)

        def body(p, carry):
            s0 = 2 * p
            issue(s0 + 1, buf1, sem1)
            drain_store(s0, buf0, sem0)
            issue(s0 + 2, buf0, sem0)
            drain_store(s0 + 1, buf1, sem1)
            return carry

        lax.fori_loop(0, nsup // 2, body, 0)
        drain_store(nsup - 1, buf0, sem0)

    return gk


def kernel(input_features, num_features, feature_index, W, b,
           bn1_gamma, bn1_beta, bn2_gamma, bn2_beta):
    n, d = input_features.shape
    m = feature_index.shape[1]
    de = num_features.shape[2]
    d2 = 2 * d
    nm = n * m

    wt_self = W[:, :d].T
    wt_nbr = W[:, d:2 * d].T
    wt_edge = W[:, 2 * d:].T
    bfull = b[None, :]
    idx32 = feature_index.astype(jnp.int32)
    ef = jnp.transpose(num_features, (1, 0, 2)).reshape(nm, de)

    bnp = 1000
    y_mat = pl.pallas_call(
        functools.partial(_prep_y_body, d=d),
        grid=(n // bnp,),
        in_specs=[
            pl.BlockSpec((bnp, d), lambda i: (i, 0)),
            pl.BlockSpec((d, d2), lambda i: (0, 0)),
        ],
        out_specs=pl.BlockSpec((bnp, d2 // 2), lambda i: (i, 0)),
        out_shape=jax.ShapeDtypeStruct((n, d2 // 2), jnp.int32),
    )(input_features, wt_nbr)
    idx_t = pl.pallas_call(
        _transpose_body,
        in_specs=[pl.BlockSpec((n, m), lambda: (0, 0))],
        out_specs=pl.BlockSpec((m, n), lambda: (0, 0)),
        out_shape=jax.ShapeDtypeStruct((m, n), jnp.int32),
    )(idx32)
    idx = idx_t.reshape(32, nm // (32 * 40), 40)

    gather = _make_gather(nm, d2 // 2, 32)
    yg = gather(y_mat, idx)

    a_mat = pl.pallas_call(
        _prep_a_body,
        grid=(n // bnp,),
        in_specs=[
            pl.BlockSpec((bnp, d), lambda i: (i, 0)),
            pl.BlockSpec((d, d2), lambda i: (0, 0)),
            pl.BlockSpec((1, d2), lambda i: (0, 0)),
        ],
        out_specs=pl.BlockSpec((bnp, d2), lambda i: (i, 0)),
        out_shape=jax.ShapeDtypeStruct((n, d2), jnp.float32),
    )(input_features, wt_self, bfull)

    bn = 2000
    nb = n // bn
    grid = (nb, m)
    row_specs = [
        pl.BlockSpec((bn, d2 // 2), lambda i, j: (j * nb + i, 0)),
        pl.BlockSpec((bn, d2), lambda i, j: (i, 0)),
        pl.BlockSpec((bn, de), lambda i, j: (j * nb + i, 0)),
        pl.BlockSpec((de, d2), lambda i, j: (0, 0)),
    ]
    acc_spec2 = pl.BlockSpec((1, d2), lambda i, j: (0, 0))
    s1_sum, s1_sq = pl.pallas_call(
        functools.partial(_stats_body, bn=bn, d=d),
        grid=grid,
        in_specs=row_specs,
        out_specs=[acc_spec2, acc_spec2],
        out_shape=[
            jax.ShapeDtypeStruct((1, d2), jnp.float32),
            jax.ShapeDtypeStruct((1, d2), jnp.float32),
        ],
    )(yg, a_mat, ef, wt_edge)

    yg2 = gather(y_mat, idx)

    mean1 = s1_sum / nm
    var1 = s1_sq / nm - mean1 * mean1
    inv1 = lax.rsqrt(var1 + EPS)
    scale1 = bn1_gamma[None, :] * inv1
    shift1 = bn1_beta[None, :] - mean1 * scale1

    acc_spec1 = pl.BlockSpec((1, d), lambda i, j: (0, 0))
    s_mat, s2_sum, s2_sq = pl.pallas_call(
        functools.partial(_pass2_body, bn=bn, m=m, d=d),
        grid=grid,
        in_specs=row_specs + [
            pl.BlockSpec((1, d2), lambda i, j: (0, 0)),
            pl.BlockSpec((1, d2), lambda i, j: (0, 0)),
        ],
        out_specs=[pl.BlockSpec((bn, d), lambda i, j: (i, 0)),
                   acc_spec1, acc_spec1],
        out_shape=[
            jax.ShapeDtypeStruct((n, d), jnp.float32),
            jax.ShapeDtypeStruct((1, d), jnp.float32),
            jax.ShapeDtypeStruct((1, d), jnp.float32),
        ],
    )(yg2, a_mat, ef, wt_edge, scale1, shift1)

    mean2 = s2_sum / n
    var2 = s2_sq / n - mean2 * mean2
    inv2 = lax.rsqrt(var2 + EPS)
    scale2 = bn2_gamma[None, :] * inv2
    shift2 = bn2_beta[None, :] - mean2 * scale2

    bnf = 1000
    out = pl.pallas_call(
        _final_body,
        grid=(n // bnf,),
        in_specs=[
            pl.BlockSpec((bnf, d), lambda i: (i, 0)),
            pl.BlockSpec((bnf, d), lambda i: (i, 0)),
            pl.BlockSpec((1, d), lambda i: (0, 0)),
            pl.BlockSpec((1, d), lambda i: (0, 0)),
        ],
        out_specs=pl.BlockSpec((bnf, d), lambda i: (i, 0)),
        out_shape=jax.ShapeDtypeStruct((n, d), jnp.float32),
    )(input_features, s_mat, scale2, shift2)
    return out


# speedup vs baseline: 1.0881x; 1.0766x over previous
import functools

import jax
import jax.numpy as jnp
from jax import lax
from jax.experimental import pallas as pl
from jax.experimental.pallas import tpu as pltpu
from jax.experimental.pallas import tpu_sc as plsc

EPS = 1e-5


def _softplus(x):
    return jnp.maximum(x, 0.0) + jnp.log(1.0 + jnp.exp(-jnp.abs(x)))


def _pack_bf16(lo_f, hi_f):
    lo = lax.bitcast_convert_type(lo_f, jnp.int32)
    hi = lax.bitcast_convert_type(hi_f, jnp.int32)
    lo_r = lo + jnp.int32(0x7FFF) + jnp.bitwise_and(
        lax.shift_right_logical(lo, 16), jnp.int32(1))
    hi_r = hi + jnp.int32(0x7FFF) + jnp.bitwise_and(
        lax.shift_right_logical(hi, 16), jnp.int32(1))
    return jnp.bitwise_or(lax.shift_right_logical(lo_r, 16),
                          jnp.bitwise_and(hi_r, jnp.int32(-65536)))


def _unpack_lo(packed):
    return lax.bitcast_convert_type(jnp.left_shift(packed, 16), jnp.float32)


def _unpack_hi(packed):
    return lax.bitcast_convert_type(
        jnp.bitwise_and(packed, jnp.int32(-65536)), jnp.float32)


def _prep_y_body(x_ref, w_ref, y_ref, *, d):
    y = jnp.dot(x_ref[...], w_ref[...], preferred_element_type=jnp.float32)
    y_ref[...] = _pack_bf16(y[:, :d], y[:, d:])


def _transpose_body(idx_ref, idxt_ref):
    idxt_ref[...] = idx_ref[...].T


def _prep_a_body(x_ref, w_ref, b_ref, a_ref):
    a_ref[...] = jnp.dot(x_ref[...], w_ref[...],
                         preferred_element_type=jnp.float32) + b_ref[...]


def _row_block(yg_ref, a_ref, ef_ref, we_ref, d):
    e = jnp.dot(ef_ref[...], we_ref[...], preferred_element_type=jnp.float32)
    yg = yg_ref[...]
    rlo = _unpack_lo(yg) + e[:, :d] + a_ref[:, :d]
    rhi = _unpack_hi(yg) + e[:, d:] + a_ref[:, d:]
    return rlo, rhi


def _stats_body(yg_ref, a_ref, ef_ref, we_ref, sum_ref, sq_ref, *, bn, d):
    @pl.when((pl.program_id(0) == 0) & (pl.program_id(1) == 0))
    def _init():
        sum_ref[...] = jnp.zeros_like(sum_ref)
        sq_ref[...] = jnp.zeros_like(sq_ref)

    rlo, rhi = _row_block(yg_ref, a_ref, ef_ref, we_ref, d)
    ones_r = jnp.ones((1, bn), jnp.float32)
    dot = functools.partial(jnp.dot, preferred_element_type=jnp.float32)
    sum_ref[:, :d] += dot(ones_r, rlo)
    sum_ref[:, d:] += dot(ones_r, rhi)
    sq_ref[:, :d] += dot(ones_r, rlo * rlo)
    sq_ref[:, d:] += dot(ones_r, rhi * rhi)


def _pass2_body(yg_ref, a_ref, ef_ref, we_ref, s1_ref, t1_ref,
                s_ref, sum_ref, sq_ref, *, bn, m, d):
    i = pl.program_id(0)
    j = pl.program_id(1)

    rlo, rhi = _row_block(yg_ref, a_ref, ef_ref, we_ref, d)
    rnlo = rlo * s1_ref[:, :d] + t1_ref[:, :d]
    rnhi = rhi * s1_ref[:, d:] + t1_ref[:, d:]
    prod = jax.nn.sigmoid(rnlo) * _softplus(rnhi)

    @pl.when(j == 0)
    def _first():
        s_ref[...] = prod

    @pl.when(j != 0)
    def _acc():
        s_ref[...] += prod

    @pl.when((i == 0) & (j == m - 1))
    def _init_stats():
        sum_ref[...] = jnp.zeros_like(sum_ref)
        sq_ref[...] = jnp.zeros_like(sq_ref)

    @pl.when(j == m - 1)
    def _stats():
        s = s_ref[...]
        ones_n = jnp.ones((1, bn), jnp.float32)
        dot = functools.partial(jnp.dot, preferred_element_type=jnp.float32)
        sum_ref[...] += dot(ones_n, s)
        sq_ref[...] += dot(ones_n, s * s)


def _final_body(x_ref, s_ref, s2_ref, t2_ref, o_ref):
    o_ref[...] = _softplus(x_ref[...] + s_ref[...] * s2_ref[...] + t2_ref[...])


def _make_gather(nrows, dcols, nw):
    rows_per_w = nrows // nw
    ch = 40
    ksub = 5
    sup = ch * ksub
    nsup = rows_per_w // sup
    nch = rows_per_w // ch
    mesh = plsc.VectorSubcoreMesh(core_axis_name="c", subcore_axis_name="s")

    @functools.partial(
        pl.kernel, mesh=mesh,
        out_type=jax.ShapeDtypeStruct((nrows, dcols), jnp.int32),
        scratch_types=[
            pltpu.VMEM((nch, ch), jnp.int32),
            pltpu.VMEM((sup, dcols), jnp.int32),
            pltpu.VMEM((sup, dcols), jnp.int32),
            pltpu.SemaphoreType.DMA,
            pltpu.SemaphoreType.DMA,
        ],
    )
    def gk(table_hbm, idx_hbm, out_hbm, idx_v, buf0, buf1, sem0, sem1):
        ncores = 2
        wid = lax.axis_index("s") * ncores + lax.axis_index("c")
        base = wid * rows_per_w
        pltpu.sync_copy(idx_hbm.at[wid], idx_v)

        def issue(sc, buf, sem):
            for k in range(ksub):
                pltpu.async_copy(table_hbm.at[idx_v.at[sc * ksub + k]],
                                 buf.at[pl.ds(k * ch, ch)], sem)

        def drain_store(sc, buf, sem):
            for k in range(ksub):
                pltpu.make_async_copy(table_hbm.at[pl.ds(0, ch)],
                                      buf.at[pl.ds(k * ch, ch)], sem).wait()
            off = pl.multiple_of(base + sc * sup, 8)
            pltpu.sync_copy(buf, out_hbm.at[pl.ds(off, sup)])

        issue(0, buf0, sem0)

        def body(p, carry):
            s0 = 2 * p
            issue(s0 + 1, buf1, sem1)
            drain_store(s0, buf0, sem0)
            issue(s0 + 2, buf0, sem0)
            drain_store(s0 + 1, buf1, sem1)
            return carry

        lax.fori_loop(0, nsup // 2, body, 0)
        drain_store(nsup - 1, buf0, sem0)

    return gk


def kernel(input_features, num_features, feature_index, W, b,
           bn1_gamma, bn1_beta, bn2_gamma, bn2_beta):
    n, d = input_features.shape
    m = feature_index.shape[1]
    de = num_features.shape[2]
    d2 = 2 * d
    nm = n * m

    wt_self = W[:, :d].T
    wt_nbr = W[:, d:2 * d].T
    wt_edge = W[:, 2 * d:].T
    bfull = b[None, :]
    idx32 = feature_index.astype(jnp.int32)
    ef = jnp.transpose(num_features, (1, 0, 2)).reshape(nm, de)

    bnp = 1000
    y_mat = pl.pallas_call(
        functools.partial(_prep_y_body, d=d),
        grid=(n // bnp,),
        in_specs=[
            pl.BlockSpec((bnp, d), lambda i: (i, 0)),
            pl.BlockSpec((d, d2), lambda i: (0, 0)),
        ],
        out_specs=pl.BlockSpec((bnp, d2 // 2), lambda i: (i, 0)),
        out_shape=jax.ShapeDtypeStruct((n, d2 // 2), jnp.int32),
    )(input_features, wt_nbr)
    idx_t = pl.pallas_call(
        _transpose_body,
        in_specs=[pl.BlockSpec((n, m), lambda: (0, 0))],
        out_specs=pl.BlockSpec((m, n), lambda: (0, 0)),
        out_shape=jax.ShapeDtypeStruct((m, n), jnp.int32),
    )(idx32)
    idx = idx_t.reshape(32, nm // (32 * 40), 40)

    gather = _make_gather(nm, d2 // 2, 32)
    yg = gather(y_mat, idx)

    a_mat = pl.pallas_call(
        _prep_a_body,
        grid=(n // bnp,),
        in_specs=[
            pl.BlockSpec((bnp, d), lambda i: (i, 0)),
            pl.BlockSpec((d, d2), lambda i: (0, 0)),
            pl.BlockSpec((1, d2), lambda i: (0, 0)),
        ],
        out_specs=pl.BlockSpec((bnp, d2), lambda i: (i, 0)),
        out_shape=jax.ShapeDtypeStruct((n, d2), jnp.float32),
    )(input_features, wt_self, bfull)

    bn = 5000
    nb = n // bn
    grid = (nb, m)
    row_specs = [
        pl.BlockSpec((bn, d2 // 2), lambda i, j: (j * nb + i, 0)),
        pl.BlockSpec((bn, d2), lambda i, j: (i, 0)),
        pl.BlockSpec((bn, de), lambda i, j: (j * nb + i, 0)),
        pl.BlockSpec((de, d2), lambda i, j: (0, 0)),
    ]
    acc_spec2 = pl.BlockSpec((1, d2), lambda i, j: (0, 0))
    s1_sum, s1_sq = pl.pallas_call(
        functools.partial(_stats_body, bn=bn, d=d),
        grid=grid,
        in_specs=row_specs,
        out_specs=[acc_spec2, acc_spec2],
        out_shape=[
            jax.ShapeDtypeStruct((1, d2), jnp.float32),
            jax.ShapeDtypeStruct((1, d2), jnp.float32),
        ],
    )(yg, a_mat, ef, wt_edge)

    yg2 = gather(y_mat, idx)

    mean1 = s1_sum / nm
    var1 = s1_sq / nm - mean1 * mean1
    inv1 = lax.rsqrt(var1 + EPS)
    scale1 = bn1_gamma[None, :] * inv1
    shift1 = bn1_beta[None, :] - mean1 * scale1

    acc_spec1 = pl.BlockSpec((1, d), lambda i, j: (0, 0))
    s_mat, s2_sum, s2_sq = pl.pallas_call(
        functools.partial(_pass2_body, bn=bn, m=m, d=d),
        grid=grid,
        in_specs=row_specs + [
            pl.BlockSpec((1, d2), lambda i, j: (0, 0)),
            pl.BlockSpec((1, d2), lambda i, j: (0, 0)),
        ],
        out_specs=[pl.BlockSpec((bn, d), lambda i, j: (i, 0)),
                   acc_spec1, acc_spec1],
        out_shape=[
            jax.ShapeDtypeStruct((n, d), jnp.float32),
            jax.ShapeDtypeStruct((1, d), jnp.float32),
            jax.ShapeDtypeStruct((1, d), jnp.float32),
        ],
    )(yg2, a_mat, ef, wt_edge, scale1, shift1)

    mean2 = s2_sum / n
    var2 = s2_sq / n - mean2 * mean2
    inv2 = lax.rsqrt(var2 + EPS)
    scale2 = bn2_gamma[None, :] * inv2
    shift2 = bn2_beta[None, :] - mean2 * scale2

    bnf = 1000
    out = pl.pallas_call(
        _final_body,
        grid=(n // bnf,),
        in_specs=[
            pl.BlockSpec((bnf, d), lambda i: (i, 0)),
            pl.BlockSpec((bnf, d), lambda i: (i, 0)),
            pl.BlockSpec((1, d), lambda i: (0, 0)),
            pl.BlockSpec((1, d), lambda i: (0, 0)),
        ],
        out_specs=pl.BlockSpec((bnf, d), lambda i: (i, 0)),
        out_shape=jax.ShapeDtypeStruct((n, d), jnp.float32),
    )(input_features, s_mat, scale2, shift2)
    return out


# speedup vs baseline: 1.0936x; 1.0050x over previous
import functools

import jax
import jax.numpy as jnp
from jax import lax
from jax.experimental import pallas as pl
from jax.experimental.pallas import tpu as pltpu
from jax.experimental.pallas import tpu_sc as plsc

EPS = 1e-5


def _softplus(x):
    return jnp.maximum(x, 0.0) + jnp.log(1.0 + jnp.exp(-jnp.abs(x)))


def _pack_bf16(lo_f, hi_f):
    lo = lax.bitcast_convert_type(lo_f, jnp.int32)
    hi = lax.bitcast_convert_type(hi_f, jnp.int32)
    lo_r = lo + jnp.int32(0x7FFF) + jnp.bitwise_and(
        lax.shift_right_logical(lo, 16), jnp.int32(1))
    hi_r = hi + jnp.int32(0x7FFF) + jnp.bitwise_and(
        lax.shift_right_logical(hi, 16), jnp.int32(1))
    return jnp.bitwise_or(lax.shift_right_logical(lo_r, 16),
                          jnp.bitwise_and(hi_r, jnp.int32(-65536)))


def _unpack_lo(packed):
    return lax.bitcast_convert_type(jnp.left_shift(packed, 16), jnp.float32)


def _unpack_hi(packed):
    return lax.bitcast_convert_type(
        jnp.bitwise_and(packed, jnp.int32(-65536)), jnp.float32)


def _prep_y_body(x_ref, w_ref, y_ref, *, d):
    y = jnp.dot(x_ref[...], w_ref[...], preferred_element_type=jnp.float32)
    y_ref[...] = _pack_bf16(y[:, :d], y[:, d:])


def _prep_a_body(x_ref, w_ref, b_ref, a_ref):
    a_ref[...] = jnp.dot(
        x_ref[...], w_ref[...], preferred_element_type=jnp.float32
    ) + b_ref[...]


def _transpose_body(idx_ref, idxt_ref):
    idxt_ref[...] = idx_ref[...].T


def _row_block(yg_ref, a_ref, ef_ref, we_ref, d):
    e = jnp.dot(ef_ref[...], we_ref[...], preferred_element_type=jnp.float32)
    yg = yg_ref[...]
    rlo = _unpack_lo(yg) + e[:, :d] + a_ref[:, :d]
    rhi = _unpack_hi(yg) + e[:, d:] + a_ref[:, d:]
    return rlo, rhi


def _stats_body(yg_ref, a_ref, ef_ref, we_ref, sum_ref, sq_ref, *, bn, d):
    @pl.when((pl.program_id(0) == 0) & (pl.program_id(1) == 0))
    def _init():
        sum_ref[...] = jnp.zeros_like(sum_ref)
        sq_ref[...] = jnp.zeros_like(sq_ref)

    rlo, rhi = _row_block(yg_ref, a_ref, ef_ref, we_ref, d)
    ones_r = jnp.ones((1, bn), jnp.float32)
    dot = functools.partial(jnp.dot, preferred_element_type=jnp.float32)
    sum_ref[:, :d] += dot(ones_r, rlo)
    sum_ref[:, d:] += dot(ones_r, rhi)
    sq_ref[:, :d] += dot(ones_r, rlo * rlo)
    sq_ref[:, d:] += dot(ones_r, rhi * rhi)


def _pass2_body(yg_ref, a_ref, ef_ref, we_ref, s1_ref, t1_ref,
                s_ref, sum_ref, sq_ref, *, bn, m, d):
    i = pl.program_id(0)
    j = pl.program_id(1)

    rlo, rhi = _row_block(yg_ref, a_ref, ef_ref, we_ref, d)
    rnlo = rlo * s1_ref[:, :d] + t1_ref[:, :d]
    rnhi = rhi * s1_ref[:, d:] + t1_ref[:, d:]
    prod = jax.nn.sigmoid(rnlo) * _softplus(rnhi)

    @pl.when(j == 0)
    def _first():
        s_ref[...] = prod

    @pl.when(j != 0)
    def _acc():
        s_ref[...] += prod

    @pl.when((i == 0) & (j == m - 1))
    def _init_stats():
        sum_ref[...] = jnp.zeros_like(sum_ref)
        sq_ref[...] = jnp.zeros_like(sq_ref)

    @pl.when(j == m - 1)
    def _stats():
        s = s_ref[...]
        ones_n = jnp.ones((1, bn), jnp.float32)
        dot = functools.partial(jnp.dot, preferred_element_type=jnp.float32)
        sum_ref[...] += dot(ones_n, s)
        sq_ref[...] += dot(ones_n, s * s)


def _final_body(x_ref, s_ref, s2_ref, t2_ref, o_ref):
    o_ref[...] = _softplus(x_ref[...] + s_ref[...] * s2_ref[...] + t2_ref[...])


def _make_gather(nrows, dcols, nw):
    rows_per_w = nrows // nw
    ch = 40
    ksub = 5
    sup = ch * ksub
    nsup = rows_per_w // sup
    nch = rows_per_w // ch
    mesh = plsc.VectorSubcoreMesh(core_axis_name="c", subcore_axis_name="s")

    @functools.partial(
        pl.kernel, mesh=mesh,
        out_type=jax.ShapeDtypeStruct((nrows, dcols), jnp.int32),
        scratch_types=[
            pltpu.VMEM((nch, ch), jnp.int32),
            pltpu.VMEM((sup, dcols), jnp.int32),
            pltpu.VMEM((sup, dcols), jnp.int32),
            pltpu.SemaphoreType.DMA,
            pltpu.SemaphoreType.DMA,
        ],
    )
    def gk(table_hbm, idx_hbm, out_hbm, idx_v, buf0, buf1, sem0, sem1):
        ncores = 2
        wid = lax.axis_index("s") * ncores + lax.axis_index("c")
        base = wid * rows_per_w
        pltpu.sync_copy(idx_hbm.at[wid], idx_v)

        def issue(sc, buf, sem):
            for k in range(ksub):
                pltpu.async_copy(table_hbm.at[idx_v.at[sc * ksub + k]],
                                 buf.at[pl.ds(k * ch, ch)], sem)

        def drain_store(sc, buf, sem):
            for k in range(ksub):
                pltpu.make_async_copy(table_hbm.at[pl.ds(0, ch)],
                                      buf.at[pl.ds(k * ch, ch)], sem).wait()
            off = pl.multiple_of(base + sc * sup, 8)
            pltpu.sync_copy(buf, out_hbm.at[pl.ds(off, sup)])

        issue(0, buf0, sem0)

        def body(p, carry):
            s0 = 2 * p
            issue(s0 + 1, buf1, sem1)
            drain_store(s0, buf0, sem0)
            issue(s0 + 2, buf0, sem0)
            drain_store(s0 + 1, buf1, sem1)
            return carry

        lax.fori_loop(0, nsup // 2, body, 0)
        drain_store(nsup - 1, buf0, sem0)

    return gk


def kernel(input_features, num_features, feature_index, W, b,
           bn1_gamma, bn1_beta, bn2_gamma, bn2_beta):
    n, d = input_features.shape
    m = feature_index.shape[1]
    de = num_features.shape[2]
    d2 = 2 * d
    nm = n * m

    wt_self = W[:, :d].T
    wt_nbr = W[:, d:2 * d].T
    wt_edge = W[:, 2 * d:].T
    brow = b[None, :]
    idx32 = feature_index.astype(jnp.int32)
    ef = jnp.transpose(num_features, (1, 0, 2)).reshape(nm, de)

    bnp = 2000
    y_mat = pl.pallas_call(
        functools.partial(_prep_y_body, d=d),
        grid=(n // bnp,),
        in_specs=[
            pl.BlockSpec((bnp, d), lambda i: (i, 0)),
            pl.BlockSpec((d, d2), lambda i: (0, 0)),
        ],
        out_specs=pl.BlockSpec((bnp, d2 // 2), lambda i: (i, 0)),
        out_shape=jax.ShapeDtypeStruct((n, d2 // 2), jnp.int32),
    )(input_features, wt_nbr)
    idx_t = pl.pallas_call(
        _transpose_body,
        in_specs=[pl.BlockSpec((n, m), lambda: (0, 0))],
        out_specs=pl.BlockSpec((m, n), lambda: (0, 0)),
        out_shape=jax.ShapeDtypeStruct((m, n), jnp.int32),
    )(idx32)
    idx = idx_t.reshape(32, nm // (32 * 40), 40)

    gather = _make_gather(nm, d2 // 2, 32)
    yg = gather(y_mat, idx)

    a_mat = pl.pallas_call(
        _prep_a_body,
        grid=(n // bnp,),
        in_specs=[
            pl.BlockSpec((bnp, d), lambda i: (i, 0)),
            pl.BlockSpec((d, d2), lambda i: (0, 0)),
            pl.BlockSpec((1, d2), lambda i: (0, 0)),
        ],
        out_specs=pl.BlockSpec((bnp, d2), lambda i: (i, 0)),
        out_shape=jax.ShapeDtypeStruct((n, d2), jnp.float32),
    )(input_features, wt_self, brow)

    bn = 5000
    nb = n // bn
    grid = (nb, m)
    row_specs = [
        pl.BlockSpec((bn, d2 // 2), lambda i, j: (j * nb + i, 0)),
        pl.BlockSpec((bn, d2), lambda i, j: (i, 0)),
        pl.BlockSpec((bn, de), lambda i, j: (j * nb + i, 0)),
        pl.BlockSpec((de, d2), lambda i, j: (0, 0)),
    ]
    acc_spec2 = pl.BlockSpec((1, d2), lambda i, j: (0, 0))
    s1_sum, s1_sq = pl.pallas_call(
        functools.partial(_stats_body, bn=bn, d=d),
        grid=grid,
        in_specs=row_specs,
        out_specs=[acc_spec2, acc_spec2],
        out_shape=[
            jax.ShapeDtypeStruct((1, d2), jnp.float32),
            jax.ShapeDtypeStruct((1, d2), jnp.float32),
        ],
    )(yg, a_mat, ef, wt_edge)

    yg2 = gather(y_mat, idx)

    mean1 = s1_sum / nm
    var1 = s1_sq / nm - mean1 * mean1
    inv1 = lax.rsqrt(var1 + EPS)
    scale1 = bn1_gamma[None, :] * inv1
    shift1 = bn1_beta[None, :] - mean1 * scale1

    acc_spec1 = pl.BlockSpec((1, d), lambda i, j: (0, 0))
    s_mat, s2_sum, s2_sq = pl.pallas_call(
        functools.partial(_pass2_body, bn=bn, m=m, d=d),
        grid=grid,
        in_specs=row_specs + [
            pl.BlockSpec((1, d2), lambda i, j: (0, 0)),
            pl.BlockSpec((1, d2), lambda i, j: (0, 0)),
        ],
        out_specs=[pl.BlockSpec((bn, d), lambda i, j: (i, 0)),
                   acc_spec1, acc_spec1],
        out_shape=[
            jax.ShapeDtypeStruct((n, d), jnp.float32),
            jax.ShapeDtypeStruct((1, d), jnp.float32),
            jax.ShapeDtypeStruct((1, d), jnp.float32),
        ],
    )(yg2, a_mat, ef, wt_edge, scale1, shift1)

    mean2 = s2_sum / n
    var2 = s2_sq / n - mean2 * mean2
    inv2 = lax.rsqrt(var2 + EPS)
    scale2 = bn2_gamma[None, :] * inv2
    shift2 = bn2_beta[None, :] - mean2 * scale2

    bnf = 1000
    out = pl.pallas_call(
        _final_body,
        grid=(n // bnf,),
        in_specs=[
            pl.BlockSpec((bnf, d), lambda i: (i, 0)),
            pl.BlockSpec((bnf, d), lambda i: (i, 0)),
            pl.BlockSpec((1, d), lambda i: (0, 0)),
            pl.BlockSpec((1, d), lambda i: (0, 0)),
        ],
        out_specs=pl.BlockSpec((bnf, d), lambda i: (i, 0)),
        out_shape=jax.ShapeDtypeStruct((n, d), jnp.float32),
    )(input_features, s_mat, scale2, shift2)
    return out
